# baseline (speedup 1.0000x reference)
import jax
import jax.numpy as jnp
from jax import lax
from jax.experimental import pallas as pl
from jax.experimental.pallas import tpu as pltpu
from jax.experimental.pallas import tpu_sc as plsc

L = 16
NC = 2
NS = 16
NW = NC * NS
B = 1024

XY_CORNERS = [(0, 0), (0, 1), (1, 0), (1, 1)]


def _make_body(n_pts, sizes):
    sx, sy, sz = sizes
    hz = sz // 2
    row_y = hz
    row_x = sy * hz
    odd_base = sx * sy * hz
    npw = n_pts // NW
    iters = npw // B
    scales = [(s - 1) * 0.5 for s in sizes]

    def body(x_hbm, y_hbm, z_hbm, table_hbm, out_hbm,
             xv, yv, zv, fx, fy, fz, idxb, dest, acc, sem):
        cid = lax.axis_index("c")
        sid = lax.axis_index("s")
        wid = cid * NS + sid
        lanes = lax.iota(jnp.int32, L)
        rep4 = lax.shift_right_logical(lanes, 2)
        col4 = jnp.bitwise_and(lanes, 3)

        @pl.loop(0, iters)
        def _it(it):
            base_pt = wid * npw + it * B
            pltpu.sync_copy(x_hbm.at[pl.ds(base_pt, B)], xv)
            pltpu.sync_copy(y_hbm.at[pl.ds(base_pt, B)], yv)
            pltpu.sync_copy(z_hbm.at[pl.ds(base_pt, B)], zv)

            @pl.loop(0, B // L)
            def _cg(g):
                s = pl.ds(g * L, L)

                def prep(v, scale, size):
                    f = v * scale + scale
                    f = jnp.maximum(f, 0.0)
                    i = jnp.minimum(f.astype(jnp.int32), size - 2)
                    return i, f - i.astype(jnp.float32)

                ix, fxv = prep(xv[s], scales[0], sx)
                iy, fyv = prep(yv[s], scales[1], sy)
                iz, fzv = prep(zv[s], scales[2], sz)
                fx[s] = fxv
                fy[s] = fyv
                fz[s] = fzv
                rowb = (jnp.bitwise_and(iz, 1) * odd_base + ix * row_x
                        + iy * row_y + lax.shift_right_logical(iz, 1))
                for k, (cx, cy) in enumerate(XY_CORNERS):
                    idxb[k, s] = rowb + (cx * row_x + cy * row_y)

            descs = [
                pltpu.async_copy(table_hbm.at[idxb.at[k]], dest.at[k], sem)
                for k in range(4)
            ]
            for d in descs:
                d.wait()

            @pl.loop(0, B // 4)
            def _acc(g2):
                ridx = rep4 + g2 * 4
                fxr = plsc.load_gather(fx, [ridx])
                fyr = plsc.load_gather(fy, [ridx])
                fzr = plsc.load_gather(fz, [ridx])
                gz0 = 1.0 - fzr
                wxy = {
                    (0, 0): (1.0 - fxr) * (1.0 - fyr),
                    (0, 1): (1.0 - fxr) * fyr,
                    (1, 0): fxr * (1.0 - fyr),
                    (1, 1): fxr * fyr,
                }
                acc_v = None
                for k, (cx, cy) in enumerate(XY_CORNERS):
                    d0 = plsc.load_gather(dest.at[k], [ridx, col4])
                    d1 = plsc.load_gather(dest.at[k], [ridx, col4 + 4])
                    contrib = wxy[(cx, cy)] * (d0 * gz0 + d1 * fzr)
                    acc_v = contrib if acc_v is None else acc_v + contrib
                plsc.store_scatter(acc, [ridx, col4], acc_v)

            pltpu.sync_copy(acc, out_hbm.at[pl.ds(base_pt, B)])

    return body


def kernel(xyz, grid):
    n_pts = xyz.shape[0]
    c = grid.shape[1]
    sizes = grid.shape[2:]
    nvox = sizes[0] * sizes[1] * sizes[2]
    x = xyz[:, 0]
    y = xyz[:, 1]
    z = xyz[:, 2]
    tflat = grid[0].reshape(c, -1).T.reshape(-1)
    t_even = tflat.reshape(nvox * c // 8, 8)
    t_odd = tflat[c:-c].reshape(nvox * c // 8 - 1, 8)
    table = jnp.concatenate([t_even, t_odd], axis=0)

    mesh = plsc.VectorSubcoreMesh(core_axis_name="c", subcore_axis_name="s")
    f = pl.kernel(
        _make_body(n_pts, sizes),
        out_type=jax.ShapeDtypeStruct((n_pts, c), jnp.float32),
        mesh=mesh,
        compiler_params=pltpu.CompilerParams(
            needs_layout_passes=False, use_tc_tiling_on_sc=False),
        scratch_types=[
            pltpu.VMEM((B,), jnp.float32),
            pltpu.VMEM((B,), jnp.float32),
            pltpu.VMEM((B,), jnp.float32),
            pltpu.VMEM((B,), jnp.float32),
            pltpu.VMEM((B,), jnp.float32),
            pltpu.VMEM((B,), jnp.float32),
            pltpu.VMEM((4, B), jnp.int32),
            pltpu.VMEM((4, B, 8), jnp.float32),
            pltpu.VMEM((B, c), jnp.float32),
            pltpu.SemaphoreType.DMA,
        ],
    )
    return f(x, y, z, table)


# speedup vs baseline: 11.1545x; 11.1545x over previous
import jax
import jax.numpy as jnp
from jax import lax
from jax.experimental import pallas as pl
from jax.experimental.pallas import tpu as pltpu
from jax.experimental.pallas import tpu_sc as plsc

L = 16
NC = 2
NS = 16
NW = NC * NS
B = 512
PF = 2048

XY_CORNERS = [(0, 0), (0, 1), (1, 0), (1, 1)]

_SC_PARAMS = pltpu.CompilerParams(
    needs_layout_passes=False, use_tc_tiling_on_sc=False)


def _detile_kernel(grid):
    _, c, sx, sy, sz = grid.shape
    rows = c * sx * sy * sz // 128
    rows_per_blk = sy * sz // 128

    def body(g_ref, o_ref):
        o_ref[...] = g_ref[...].reshape(o_ref.shape)

    return pl.pallas_call(
        body,
        grid=(c * sx,),
        in_specs=[pl.BlockSpec((1, 1, 1, sy, sz),
                               lambda i: (0, i // 256, i % 256, 0, 0))],
        out_specs=pl.BlockSpec((rows_per_blk, 128), lambda i: (i, 0)),
        out_shape=jax.ShapeDtypeStruct((rows, 128), jnp.float32),
    )(grid)


def _make_pack_body(nvox, c):
    n_chunks = nvox // NW // PF
    rows_out = PF * c // 16
    cin_rows = PF // 128
    ch_rows = nvox // 128

    def body(gflat_hbm, table_hbm, cin, cout, sem):
        wid = lax.axis_index("c") * NS + lax.axis_index("s")
        lanes = lax.iota(jnp.int32, L)
        rep4 = lax.shift_right_logical(lanes, 2)
        col4 = jnp.bitwise_and(lanes, 3)

        @pl.loop(0, n_chunks)
        def _chunk(j):
            f0 = wid * (nvox // NW) + j * PF
            for ch in range(c):
                pltpu.sync_copy(
                    gflat_hbm.at[pl.ds(ch * ch_rows + f0 // 128, cin_rows)],
                    cin.at[ch])

            @pl.loop(0, rows_out)
            def _row(r):
                vi = r * 4 + rep4
                v = plsc.load_gather(
                    cin, [col4, lax.shift_right_logical(vi, 7),
                          jnp.bitwise_and(vi, 127)])
                cout[r, :] = v

            pltpu.sync_copy(cout, table_hbm.at[pl.ds(f0 // 4, rows_out)])

    return body


def _make_main_body(n_pts, c, sizes):
    sx, sy, sz = sizes
    qz = sz // 4
    row_y = qz
    row_x = sy * qz
    npw = n_pts // NW
    iters = npw // B
    xyz_rows = B * 3 // 128
    out_rows = B * c // 128
    scales = [(s - 1) * 0.5 for s in sizes]

    def body(xyz_hbm, table_hbm, out_hbm,
             xyzv, fx, fy, fz, lb0, lb1, idxb, dest, acc, sem):
        wid = lax.axis_index("c") * NS + lax.axis_index("s")
        lanes = lax.iota(jnp.int32, L)
        rep4 = lax.shift_right_logical(lanes, 2)
        col4 = jnp.bitwise_and(lanes, 3)

        @pl.loop(0, iters)
        def _it(it):
            tile = wid * iters + it
            pltpu.sync_copy(xyz_hbm.at[pl.ds(tile * xyz_rows, xyz_rows)], xyzv)

            @pl.loop(0, B // L)
            def _cg(g):
                s = pl.ds(g * L, L)
                e = (g * L + lanes) * 3

                def coord(d):
                    ei = e + d
                    return plsc.load_gather(
                        xyzv, [lax.shift_right_logical(ei, 7),
                               jnp.bitwise_and(ei, 127)])

                def prep(v, scale, size):
                    f = v * scale + scale
                    f = jnp.maximum(f, 0.0)
                    i = jnp.minimum(f.astype(jnp.int32), size - 2)
                    return i, f - i.astype(jnp.float32)

                ix, fxv = prep(coord(0), scales[0], sx)
                iy, fyv = prep(coord(1), scales[1], sy)
                iz, fzv = prep(coord(2), scales[2], sz)
                fx[s] = fxv
                fy[s] = fyv
                fz[s] = fzv
                iz1 = iz + 1
                lb0[s] = jnp.bitwise_and(iz, 3) * 4
                lb1[s] = jnp.bitwise_and(iz1, 3) * 4
                h0 = (ix * row_x + iy * row_y
                      + lax.shift_right_logical(iz, 2))
                h1 = (ix * row_x + iy * row_y
                      + lax.shift_right_logical(iz1, 2))
                for k, (cx, cy) in enumerate(XY_CORNERS):
                    off = cx * row_x + cy * row_y
                    idxb[k, s] = h0 + off
                    idxb[4 + k, s] = h1 + off

            descs = [
                pltpu.async_copy(table_hbm.at[idxb.at[k]], dest.at[k], sem)
                for k in range(8)
            ]
            for d in descs:
                d.wait()

            @pl.loop(0, B // 4)
            def _acc(g2):
                ridx = rep4 + g2 * 4
                fxr = plsc.load_gather(fx, [ridx])
                fyr = plsc.load_gather(fy, [ridx])
                fzr = plsc.load_gather(fz, [ridx])
                l0r = plsc.load_gather(lb0, [ridx]) + col4
                l1r = plsc.load_gather(lb1, [ridx]) + col4
                gz0 = 1.0 - fzr
                wxy = {
                    (0, 0): (1.0 - fxr) * (1.0 - fyr),
                    (0, 1): (1.0 - fxr) * fyr,
                    (1, 0): fxr * (1.0 - fyr),
                    (1, 1): fxr * fyr,
                }
                acc_v = None
                for k, (cx, cy) in enumerate(XY_CORNERS):
                    d0 = plsc.load_gather(dest.at[k], [ridx, l0r])
                    d1 = plsc.load_gather(dest.at[4 + k], [ridx, l1r])
                    contrib = wxy[(cx, cy)] * (d0 * gz0 + d1 * fzr)
                    acc_v = contrib if acc_v is None else acc_v + contrib
                eo = ridx * c + col4
                plsc.store_scatter(
                    acc, [lax.shift_right_logical(eo, 7),
                          jnp.bitwise_and(eo, 127)], acc_v)

            pltpu.sync_copy(acc, out_hbm.at[pl.ds(tile * out_rows, out_rows)])

    return body


def kernel(xyz, grid):
    n_pts = xyz.shape[0]
    c = grid.shape[1]
    sizes = grid.shape[2:]
    nvox = sizes[0] * sizes[1] * sizes[2]
    mesh = plsc.VectorSubcoreMesh(core_axis_name="c", subcore_axis_name="s")

    gflat = _detile_kernel(grid)

    pack = pl.kernel(
        _make_pack_body(nvox, c),
        out_type=jax.ShapeDtypeStruct((nvox // 4, 16), jnp.float32),
        mesh=mesh,
        compiler_params=_SC_PARAMS,
        scratch_types=[
            pltpu.VMEM((c, PF // 128, 128), jnp.float32),
            pltpu.VMEM((PF * c // 16, 16), jnp.float32),
            pltpu.SemaphoreType.DMA,
        ],
    )
    table = pack(gflat)

    xyz128 = xyz.reshape(n_pts * 3 // 128, 128)
    main = pl.kernel(
        _make_main_body(n_pts, c, sizes),
        out_type=jax.ShapeDtypeStruct((n_pts * c // 128, 128), jnp.float32),
        mesh=mesh,
        compiler_params=_SC_PARAMS,
        scratch_types=[
            pltpu.VMEM((B * 3 // 128, 128), jnp.float32),
            pltpu.VMEM((B,), jnp.float32),
            pltpu.VMEM((B,), jnp.float32),
            pltpu.VMEM((B,), jnp.float32),
            pltpu.VMEM((B,), jnp.int32),
            pltpu.VMEM((B,), jnp.int32),
            pltpu.VMEM((8, B), jnp.int32),
            pltpu.VMEM((8, B, 16), jnp.float32),
            pltpu.VMEM((B * 4 // 128, 128), jnp.float32),
            pltpu.SemaphoreType.DMA,
        ],
    )
    out128 = main(xyz128, table)
    return out128.reshape(n_pts, c)


# speedup vs baseline: 18.1015x; 1.6228x over previous
import jax
import jax.numpy as jnp
from jax import lax
from jax.experimental import pallas as pl
from jax.experimental.pallas import tpu as pltpu
from jax.experimental.pallas import tpu_sc as plsc

L = 16
NC = 2
NS = 16
NW = NC * NS
B = 512
PF = 4096

XY_CORNERS = [(0, 0), (0, 1), (1, 0), (1, 1)]

_SC_PARAMS = pltpu.CompilerParams(
    needs_layout_passes=False, use_tc_tiling_on_sc=False)


def _detile_kernel(grid):
    _, c, sx, sy, sz = grid.shape
    rows = c * sx * sy * sz // 128
    rows_per_blk = sy * sz // 128

    def body(g_ref, o_ref):
        o_ref[...] = g_ref[...].reshape(o_ref.shape)

    return pl.pallas_call(
        body,
        grid=(c * sx,),
        in_specs=[pl.BlockSpec((1, 1, 1, sy, sz),
                               lambda i: (0, i // 256, i % 256, 0, 0))],
        out_specs=pl.BlockSpec((rows_per_blk, 128), lambda i: (i, 0)),
        out_shape=jax.ShapeDtypeStruct((rows, 128), jnp.float32),
    )(grid)


def _make_pack_body(nvox, c):
    npw = nvox // NW
    n_chunks = npw // PF
    rows_out = PF * c // 16
    cin_rows = PF // 128
    ch_rows = nvox // 128

    def body(gflat_hbm, table_hbm, cin, cout,
             lsem0, lsem1, ssem0, ssem1):
        wid = lax.axis_index("c") * NS + lax.axis_index("s")
        lanes = lax.iota(jnp.int32, L)
        rep4 = lax.shift_right_logical(lanes, 2)
        col4 = jnp.bitwise_and(lanes, 3)
        lsems = (lsem0, lsem1)
        ssems = (ssem0, ssem1)

        def load_descs(j, slot):
            f0 = wid * npw + j * PF
            return [
                pltpu.make_async_copy(
                    gflat_hbm.at[pl.ds(ch * ch_rows + f0 // 128, cin_rows)],
                    cin.at[slot, ch], lsems[slot])
                for ch in range(c)
            ]

        def store_desc(j, slot):
            f0 = wid * npw + j * PF
            return pltpu.make_async_copy(
                cout.at[slot], table_hbm.at[pl.ds(f0 // 4, rows_out)],
                ssems[slot])

        for d in load_descs(0, 0):
            d.start()

        @pl.loop(0, n_chunks // 2)
        def _pair(p):
            for par in (0, 1):
                j = p * 2 + par
                for d in load_descs(j, par):
                    d.wait()

                @pl.when(j + 1 < n_chunks)
                def _():
                    for d in load_descs(j + 1, 1 - par):
                        d.start()

                @pl.when(j >= 2)
                def _():
                    store_desc(j - 2, par).wait()

                @pl.loop(0, rows_out, unroll=8)
                def _row(r):
                    vi = r * 4 + rep4
                    v = plsc.load_gather(
                        cin.at[par], [col4, lax.shift_right_logical(vi, 7),
                                      jnp.bitwise_and(vi, 127)])
                    cout[par, r, :] = v

                store_desc(j, par).start()

        store_desc(n_chunks - 2, 0).wait()
        store_desc(n_chunks - 1, 1).wait()

    return body


def _make_main_body(n_pts, c, sizes):
    sx, sy, sz = sizes
    qz = sz // 4
    row_y = qz
    row_x = sy * qz
    npw = n_pts // NW
    iters = npw // B
    xyz_rows = B * 3 // 128
    out_rows = B // 128
    ch_out_rows = n_pts // 128
    scales = [(s - 1) * 0.5 for s in sizes]

    def body(xyz_hbm, table_hbm, out_hbm,
             xyzv, fx, fy, fz, lb0, lb1, idxb, dest, acc, gsem, osem):
        wid = lax.axis_index("c") * NS + lax.axis_index("s")
        lanes = lax.iota(jnp.int32, L)
        rep4 = lax.shift_right_logical(lanes, 2)
        col4 = jnp.bitwise_and(lanes, 3)

        def out_descs(tile):
            return [
                pltpu.make_async_copy(
                    acc.at[pl.ds(cc * out_rows, out_rows)],
                    out_hbm.at[pl.ds(cc * ch_out_rows + tile * out_rows,
                                     out_rows)],
                    osem)
                for cc in range(c)
            ]

        @pl.loop(0, iters)
        def _it(it):
            tile = wid * iters + it
            pltpu.sync_copy(xyz_hbm.at[pl.ds(tile * xyz_rows, xyz_rows)], xyzv)

            @pl.loop(0, B // L, unroll=4)
            def _cg(g):
                s = pl.ds(g * L, L)
                e = (g * L + lanes) * 3

                def coord(d):
                    ei = e + d
                    return plsc.load_gather(
                        xyzv, [lax.shift_right_logical(ei, 7),
                               jnp.bitwise_and(ei, 127)])

                def prep(v, scale, size):
                    f = v * scale + scale
                    f = jnp.maximum(f, 0.0)
                    i = jnp.minimum(f.astype(jnp.int32), size - 2)
                    return i, f - i.astype(jnp.float32)

                ix, fxv = prep(coord(0), scales[0], sx)
                iy, fyv = prep(coord(1), scales[1], sy)
                iz, fzv = prep(coord(2), scales[2], sz)
                fx[s] = fxv
                fy[s] = fyv
                fz[s] = fzv
                iz1 = iz + 1
                lb0[s] = jnp.bitwise_and(iz, 3) * 4
                lb1[s] = jnp.bitwise_and(iz1, 3) * 4
                xyb = ix * row_x + iy * row_y
                h0 = xyb + lax.shift_right_logical(iz, 2)
                h1 = xyb + lax.shift_right_logical(iz1, 2)
                for k, (cx, cy) in enumerate(XY_CORNERS):
                    off = cx * row_x + cy * row_y
                    idxb[k, s] = h0 + off
                    idxb[4 + k, s] = h1 + off

            descs = [
                pltpu.async_copy(table_hbm.at[idxb.at[k]], dest.at[k], gsem)
                for k in range(8)
            ]
            @pl.when(it > 0)
            def _():
                for d in out_descs(tile - 1):
                    d.wait()
            for d in descs:
                d.wait()

            @pl.loop(0, B // 4, unroll=4)
            def _acc(g2):
                ridx = rep4 + g2 * 4
                fxr = plsc.load_gather(fx, [ridx])
                fyr = plsc.load_gather(fy, [ridx])
                fzr = plsc.load_gather(fz, [ridx])
                l0r = plsc.load_gather(lb0, [ridx]) + col4
                l1r = plsc.load_gather(lb1, [ridx]) + col4
                gz0 = 1.0 - fzr
                wxy = {
                    (0, 0): (1.0 - fxr) * (1.0 - fyr),
                    (0, 1): (1.0 - fxr) * fyr,
                    (1, 0): fxr * (1.0 - fyr),
                    (1, 1): fxr * fyr,
                }
                acc_v = None
                for k, (cx, cy) in enumerate(XY_CORNERS):
                    d0 = plsc.load_gather(dest.at[k], [ridx, l0r])
                    d1 = plsc.load_gather(dest.at[4 + k], [ridx, l1r])
                    contrib = wxy[(cx, cy)] * (d0 * gz0 + d1 * fzr)
                    acc_v = contrib if acc_v is None else acc_v + contrib
                eo = col4 * B + ridx
                plsc.store_scatter(
                    acc, [lax.shift_right_logical(eo, 7),
                          jnp.bitwise_and(eo, 127)], acc_v)

            for d in out_descs(tile):
                d.start()

        for d in out_descs(wid * iters + iters - 1):
            d.wait()

    return body


def kernel(xyz, grid):
    n_pts = xyz.shape[0]
    c = grid.shape[1]
    sizes = grid.shape[2:]
    nvox = sizes[0] * sizes[1] * sizes[2]
    mesh = plsc.VectorSubcoreMesh(core_axis_name="c", subcore_axis_name="s")

    gflat = _detile_kernel(grid)

    pack = pl.kernel(
        _make_pack_body(nvox, c),
        out_type=jax.ShapeDtypeStruct((nvox // 4, 16), jnp.float32),
        mesh=mesh,
        compiler_params=_SC_PARAMS,
        scratch_types=[
            pltpu.VMEM((2, c, PF // 128, 128), jnp.float32),
            pltpu.VMEM((2, PF * c // 16, 16), jnp.float32),
            pltpu.SemaphoreType.DMA,
            pltpu.SemaphoreType.DMA,
            pltpu.SemaphoreType.DMA,
            pltpu.SemaphoreType.DMA,
        ],
    )
    table = pack(gflat)

    xyz128 = xyz.reshape(n_pts * 3 // 128, 128)
    main = pl.kernel(
        _make_main_body(n_pts, c, sizes),
        out_type=jax.ShapeDtypeStruct((n_pts * c // 128, 128), jnp.float32),
        mesh=mesh,
        compiler_params=_SC_PARAMS,
        scratch_types=[
            pltpu.VMEM((B * 3 // 128, 128), jnp.float32),
            pltpu.VMEM((B,), jnp.float32),
            pltpu.VMEM((B,), jnp.float32),
            pltpu.VMEM((B,), jnp.float32),
            pltpu.VMEM((B,), jnp.int32),
            pltpu.VMEM((B,), jnp.int32),
            pltpu.VMEM((8, B), jnp.int32),
            pltpu.VMEM((8, B, 16), jnp.float32),
            pltpu.VMEM((B * 4 // 128, 128), jnp.float32),
            pltpu.SemaphoreType.DMA,
            pltpu.SemaphoreType.DMA,
        ],
    )
    out128 = main(xyz128, table)
    return out128.reshape(c, n_pts).T


# speedup vs baseline: 22.0812x; 1.2199x over previous
import jax
import jax.numpy as jnp
from jax import lax
from jax.experimental import pallas as pl
from jax.experimental.pallas import tpu as pltpu
from jax.experimental.pallas import tpu_sc as plsc

L = 16
NC = 2
NS = 16
NW = NC * NS
B = 512
PF = 4096

XY_CORNERS = [(0, 0), (0, 1), (1, 0), (1, 1)]

_SC_PARAMS = pltpu.CompilerParams(
    needs_layout_passes=False, use_tc_tiling_on_sc=False)


def _detile_kernel(grid):
    _, c, sx, sy, sz = grid.shape
    rows = c * sx * sy * sz // 128
    rows_per_blk = sy * sz // 128

    def body(g_ref, o_ref):
        o_ref[...] = g_ref[...].reshape(o_ref.shape)

    return pl.pallas_call(
        body,
        grid=(c * sx,),
        in_specs=[pl.BlockSpec((1, 1, 1, sy, sz),
                               lambda i: (0, i // 256, i % 256, 0, 0))],
        out_specs=pl.BlockSpec((rows_per_blk, 128), lambda i: (i, 0)),
        out_shape=jax.ShapeDtypeStruct((rows, 128), jnp.float32),
    )(grid)


def _make_pack_body(nvox, c):
    npw = nvox // NW
    n_chunks = npw // PF
    rows_out = PF * c // 16
    cin_rows = PF // 128
    ch_rows = nvox // 128

    def body(gflat_hbm, table_hbm, cin, cout,
             lsem0, lsem1, ssem0, ssem1):
        wid = lax.axis_index("c") * NS + lax.axis_index("s")
        lanes = lax.iota(jnp.int32, L)
        rep4 = lax.shift_right_logical(lanes, 2)
        col4 = jnp.bitwise_and(lanes, 3)
        lsems = (lsem0, lsem1)
        ssems = (ssem0, ssem1)

        def load_descs(j, slot):
            f0 = wid * npw + j * PF
            return [
                pltpu.make_async_copy(
                    gflat_hbm.at[pl.ds(ch * ch_rows + f0 // 128, cin_rows)],
                    cin.at[slot, ch], lsems[slot])
                for ch in range(c)
            ]

        def store_desc(j, slot):
            f0 = wid * npw + j * PF
            return pltpu.make_async_copy(
                cout.at[slot], table_hbm.at[pl.ds(f0 // 4, rows_out)],
                ssems[slot])

        for d in load_descs(0, 0):
            d.start()

        @pl.loop(0, n_chunks // 2)
        def _pair(p):
            for par in (0, 1):
                j = p * 2 + par
                for d in load_descs(j, par):
                    d.wait()

                @pl.when(j + 1 < n_chunks)
                def _():
                    for d in load_descs(j + 1, 1 - par):
                        d.start()

                @pl.when(j >= 2)
                def _():
                    store_desc(j - 2, par).wait()

                @pl.loop(0, rows_out, unroll=8)
                def _row(r):
                    vi = r * 4 + rep4
                    v = plsc.load_gather(
                        cin.at[par], [col4, lax.shift_right_logical(vi, 7),
                                      jnp.bitwise_and(vi, 127)])
                    cout[par, r, :] = v

                store_desc(j, par).start()

        store_desc(n_chunks - 2, 0).wait()
        store_desc(n_chunks - 1, 1).wait()

    return body


def _make_main_body(n_pts, c, sizes):
    sx, sy, sz = sizes
    qz = sz // 4
    row_y = qz
    row_x = sy * qz
    npw = n_pts // NW
    iters = npw // B
    crd_rows = B // 128
    ch_crd_rows = n_pts // 128
    out_rows = B // 128
    ch_out_rows = n_pts // 128
    scales = [(s - 1) * 0.5 for s in sizes]

    def body(xyz_hbm, table_hbm, out_hbm,
             xv, yv, zv, fx, fy, fz, lb0, lb1, idxb, dest, acc, gsem, osem):
        wid = lax.axis_index("c") * NS + lax.axis_index("s")
        lanes = lax.iota(jnp.int32, L)
        rep4 = lax.shift_right_logical(lanes, 2)
        col4 = jnp.bitwise_and(lanes, 3)

        def out_descs(tile):
            return [
                pltpu.make_async_copy(
                    acc.at[pl.ds(cc * out_rows, out_rows)],
                    out_hbm.at[pl.ds(cc * ch_out_rows + tile * out_rows,
                                     out_rows)],
                    osem)
                for cc in range(c)
            ]

        @pl.loop(0, iters)
        def _it(it):
            tile = wid * iters + it
            for d, buf in ((0, xv), (1, yv), (2, zv)):
                pltpu.sync_copy(
                    xyz_hbm.at[pl.ds(d * ch_crd_rows + tile * crd_rows,
                                     crd_rows)], buf)

            @pl.loop(0, B // L, unroll=4)
            def _cg(g):
                s = pl.ds(g * L, L)
                r = lax.shift_right_logical(g, 3)
                o = jnp.bitwise_and(g, 7) * L

                def prep(v, scale, size):
                    f = v * scale + scale
                    f = jnp.maximum(f, 0.0)
                    i = jnp.minimum(f.astype(jnp.int32), size - 2)
                    return i, f - i.astype(jnp.float32)

                ix, fxv = prep(xv[r, pl.ds(o, L)], scales[0], sx)
                iy, fyv = prep(yv[r, pl.ds(o, L)], scales[1], sy)
                iz, fzv = prep(zv[r, pl.ds(o, L)], scales[2], sz)
                fx[s] = fxv
                fy[s] = fyv
                fz[s] = fzv
                iz1 = iz + 1
                lb0[s] = jnp.bitwise_and(iz, 3) * 4
                lb1[s] = jnp.bitwise_and(iz1, 3) * 4
                xyb = ix * row_x + iy * row_y
                h0 = xyb + lax.shift_right_logical(iz, 2)
                h1 = xyb + lax.shift_right_logical(iz1, 2)
                for k, (cx, cy) in enumerate(XY_CORNERS):
                    off = cx * row_x + cy * row_y
                    idxb[k, s] = h0 + off
                    idxb[4 + k, s] = h1 + off

            descs = [
                pltpu.async_copy(table_hbm.at[idxb.at[k]], dest.at[k], gsem)
                for k in range(8)
            ]
            @pl.when(it > 0)
            def _():
                for d in out_descs(tile - 1):
                    d.wait()
            for d in descs:
                d.wait()

            @pl.loop(0, B // 4, unroll=4)
            def _acc(g2):
                ridx = rep4 + g2 * 4
                fxr = plsc.load_gather(fx, [ridx])
                fyr = plsc.load_gather(fy, [ridx])
                fzr = plsc.load_gather(fz, [ridx])
                l0r = plsc.load_gather(lb0, [ridx]) + col4
                l1r = plsc.load_gather(lb1, [ridx]) + col4
                gz0 = 1.0 - fzr
                wxy = {
                    (0, 0): (1.0 - fxr) * (1.0 - fyr),
                    (0, 1): (1.0 - fxr) * fyr,
                    (1, 0): fxr * (1.0 - fyr),
                    (1, 1): fxr * fyr,
                }
                acc_v = None
                for k, (cx, cy) in enumerate(XY_CORNERS):
                    d0 = plsc.load_gather(dest.at[k], [ridx, l0r])
                    d1 = plsc.load_gather(dest.at[4 + k], [ridx, l1r])
                    contrib = wxy[(cx, cy)] * (d0 * gz0 + d1 * fzr)
                    acc_v = contrib if acc_v is None else acc_v + contrib
                eo = col4 * B + ridx
                plsc.store_scatter(
                    acc, [lax.shift_right_logical(eo, 7),
                          jnp.bitwise_and(eo, 127)], acc_v)

            for d in out_descs(tile):
                d.start()

        for d in out_descs(wid * iters + iters - 1):
            d.wait()

    return body


def kernel(xyz, grid):
    n_pts = xyz.shape[0]
    c = grid.shape[1]
    sizes = grid.shape[2:]
    nvox = sizes[0] * sizes[1] * sizes[2]
    mesh = plsc.VectorSubcoreMesh(core_axis_name="c", subcore_axis_name="s")

    gflat = _detile_kernel(grid)

    pack = pl.kernel(
        _make_pack_body(nvox, c),
        out_type=jax.ShapeDtypeStruct((nvox // 4, 16), jnp.float32),
        mesh=mesh,
        compiler_params=_SC_PARAMS,
        scratch_types=[
            pltpu.VMEM((2, c, PF // 128, 128), jnp.float32),
            pltpu.VMEM((2, PF * c // 16, 16), jnp.float32),
            pltpu.SemaphoreType.DMA,
            pltpu.SemaphoreType.DMA,
            pltpu.SemaphoreType.DMA,
            pltpu.SemaphoreType.DMA,
        ],
    )
    table = pack(gflat)

    xyz128 = xyz.T.reshape(n_pts * 3 // 128, 128)
    main = pl.kernel(
        _make_main_body(n_pts, c, sizes),
        out_type=jax.ShapeDtypeStruct((n_pts * c // 128, 128), jnp.float32),
        mesh=mesh,
        compiler_params=_SC_PARAMS,
        scratch_types=[
            pltpu.VMEM((B // 128, 128), jnp.float32),
            pltpu.VMEM((B // 128, 128), jnp.float32),
            pltpu.VMEM((B // 128, 128), jnp.float32),
            pltpu.VMEM((B,), jnp.float32),
            pltpu.VMEM((B,), jnp.float32),
            pltpu.VMEM((B,), jnp.float32),
            pltpu.VMEM((B,), jnp.int32),
            pltpu.VMEM((B,), jnp.int32),
            pltpu.VMEM((8, B), jnp.int32),
            pltpu.VMEM((8, B, 16), jnp.float32),
            pltpu.VMEM((B * 4 // 128, 128), jnp.float32),
            pltpu.SemaphoreType.DMA,
            pltpu.SemaphoreType.DMA,
        ],
    )
    out128 = main(xyz128, table)
    return out128.reshape(c, n_pts).T


# speedup vs baseline: 31.1534x; 1.4109x over previous
import jax
import jax.numpy as jnp
from jax import lax
from jax.experimental import pallas as pl
from jax.experimental.pallas import tpu as pltpu
from jax.experimental.pallas import tpu_sc as plsc

L = 16
NC = 2
NS = 16
NW = NC * NS
B = 256
PF = 4096

XY_CORNERS = [(0, 0), (0, 1), (1, 0), (1, 1)]

_SC_PARAMS = pltpu.CompilerParams(
    needs_layout_passes=False, use_tc_tiling_on_sc=False)


def _detile_kernel(grid):
    _, c, sx, sy, sz = grid.shape
    rows = c * sx * sy * sz // 128
    rows_per_blk = sy * sz // 128

    xb = 4
    def body(g_ref, o_ref):
        o_ref[...] = g_ref[...].reshape(o_ref.shape)

    return pl.pallas_call(
        body,
        grid=(c * sx // xb,),
        in_specs=[pl.BlockSpec((1, 1, xb, sy, sz),
                               lambda i: (0, i // (256 // xb),
                                          i % (256 // xb), 0, 0))],
        out_specs=pl.BlockSpec((rows_per_blk * xb, 128), lambda i: (i, 0)),
        out_shape=jax.ShapeDtypeStruct((rows, 128), jnp.float32),
    )(grid)


def _make_pack_body(nvox, c):
    npw = nvox // NW
    n_chunks = npw // PF
    rows_out = PF * c // 16
    cin_rows = PF // 128
    ch_rows = nvox // 128

    def body(gflat_hbm, table_hbm, cin, cout,
             lsem0, lsem1, ssem0, ssem1):
        wid = lax.axis_index("c") * NS + lax.axis_index("s")
        lanes = lax.iota(jnp.int32, L)
        rep4 = lax.shift_right_logical(lanes, 2)
        col4 = jnp.bitwise_and(lanes, 3)
        lsems = (lsem0, lsem1)
        ssems = (ssem0, ssem1)

        def load_descs(j, slot):
            f0 = wid * npw + j * PF
            return [
                pltpu.make_async_copy(
                    gflat_hbm.at[pl.ds(ch * ch_rows + f0 // 128, cin_rows)],
                    cin.at[slot, ch], lsems[slot])
                for ch in range(c)
            ]

        def store_desc(j, slot):
            f0 = wid * npw + j * PF
            return pltpu.make_async_copy(
                cout.at[slot], table_hbm.at[pl.ds(f0 // 4, rows_out)],
                ssems[slot])

        for d in load_descs(0, 0):
            d.start()

        @pl.loop(0, n_chunks // 2)
        def _pair(p):
            for par in (0, 1):
                j = p * 2 + par
                for d in load_descs(j, par):
                    d.wait()

                @pl.when(j + 1 < n_chunks)
                def _():
                    for d in load_descs(j + 1, 1 - par):
                        d.start()

                @pl.when(j >= 2)
                def _():
                    store_desc(j - 2, par).wait()

                @pl.loop(0, rows_out, unroll=8)
                def _row(r):
                    vi = r * 4 + rep4
                    v = plsc.load_gather(
                        cin.at[par], [col4, lax.shift_right_logical(vi, 7),
                                      jnp.bitwise_and(vi, 127)])
                    cout[par, r, :] = v

                store_desc(j, par).start()

        store_desc(n_chunks - 2, 0).wait()
        store_desc(n_chunks - 1, 1).wait()

    return body


def _make_main_body(n_pts, c, sizes):
    sx, sy, sz = sizes
    qz = sz // 4
    row_y = qz
    row_x = sy * qz
    npw = n_pts // NW
    iters = npw // B
    crd_rows = B // 128
    ch_crd_rows = n_pts // 128
    out_rows = B // 128
    ch_out_rows = n_pts // 128
    scales = [(s - 1) * 0.5 for s in sizes]

    def body(xyz_hbm, table_hbm, out_hbm,
             xv, yv, zv, fx, fy, fz, lb0, lb1, idxb, dest, acc,
             gsem0, gsem1, osem0, osem1):
        wid = lax.axis_index("c") * NS + lax.axis_index("s")
        lanes = lax.iota(jnp.int32, L)
        rep4 = lax.shift_right_logical(lanes, 2)
        col4 = jnp.bitwise_and(lanes, 3)
        gsems = (gsem0, gsem1)
        osems = (osem0, osem1)

        def load_xyz(it, sl):
            tile = wid * iters + it
            for d, buf in ((0, xv), (1, yv), (2, zv)):
                pltpu.sync_copy(
                    xyz_hbm.at[pl.ds(d * ch_crd_rows + tile * crd_rows,
                                     crd_rows)], buf.at[sl])

        def stage1(sl):
            @pl.loop(0, B // L, unroll=4)
            def _cg(g):
                s = pl.ds(g * L, L)
                r = lax.shift_right_logical(g, 3)
                o = jnp.bitwise_and(g, 7) * L

                def prep(v, scale, size):
                    f = v * scale + scale
                    f = jnp.maximum(f, 0.0)
                    i = jnp.minimum(f.astype(jnp.int32), size - 2)
                    return i, f - i.astype(jnp.float32)

                ix, fxv = prep(xv[sl, r, pl.ds(o, L)], scales[0], sx)
                iy, fyv = prep(yv[sl, r, pl.ds(o, L)], scales[1], sy)
                iz, fzv = prep(zv[sl, r, pl.ds(o, L)], scales[2], sz)
                fx[sl, s] = fxv
                fy[sl, s] = fyv
                fz[sl, s] = fzv
                iz1 = iz + 1
                lb0[sl, s] = jnp.bitwise_and(iz, 3) * 4
                lb1[sl, s] = jnp.bitwise_and(iz1, 3) * 4
                xyb = ix * row_x + iy * row_y
                h0 = xyb + lax.shift_right_logical(iz, 2)
                h1 = xyb + lax.shift_right_logical(iz1, 2)
                for k, (cx, cy) in enumerate(XY_CORNERS):
                    off = cx * row_x + cy * row_y
                    idxb[sl, k, s] = h0 + off
                    idxb[sl, 4 + k, s] = h1 + off

        def gather_descs(sl):
            return [
                pltpu.make_async_copy(table_hbm.at[idxb.at[sl, k]],
                                      dest.at[sl, k], gsems[sl])
                for k in range(8)
            ]

        def out_descs(it, sl):
            tile = wid * iters + it
            return [
                pltpu.make_async_copy(
                    acc.at[sl, pl.ds(cc * out_rows, out_rows)],
                    out_hbm.at[pl.ds(cc * ch_out_rows + tile * out_rows,
                                     out_rows)],
                    osems[sl])
                for cc in range(c)
            ]

        def stage3(sl):
            @pl.loop(0, B // 4, unroll=4)
            def _acc(g2):
                ridx = rep4 + g2 * 4
                fxr = plsc.load_gather(fx.at[sl], [ridx])
                fyr = plsc.load_gather(fy.at[sl], [ridx])
                fzr = plsc.load_gather(fz.at[sl], [ridx])
                l0r = plsc.load_gather(lb0.at[sl], [ridx]) + col4
                l1r = plsc.load_gather(lb1.at[sl], [ridx]) + col4
                gz0 = 1.0 - fzr
                wxy = {
                    (0, 0): (1.0 - fxr) * (1.0 - fyr),
                    (0, 1): (1.0 - fxr) * fyr,
                    (1, 0): fxr * (1.0 - fyr),
                    (1, 1): fxr * fyr,
                }
                acc_v = None
                for k, (cx, cy) in enumerate(XY_CORNERS):
                    d0 = plsc.load_gather(dest.at[sl, k], [ridx, l0r])
                    d1 = plsc.load_gather(dest.at[sl, 4 + k], [ridx, l1r])
                    contrib = wxy[(cx, cy)] * (d0 * gz0 + d1 * fzr)
                    acc_v = contrib if acc_v is None else acc_v + contrib
                eo = col4 * B + ridx
                plsc.store_scatter(
                    acc.at[sl], [lax.shift_right_logical(eo, 7),
                                 jnp.bitwise_and(eo, 127)], acc_v)

        load_xyz(0, 0)
        stage1(0)
        for d in gather_descs(0):
            d.start()

        @pl.loop(0, iters // 2)
        def _pair(p):
            for par in (0, 1):
                it = p * 2 + par

                @pl.when(it + 1 < iters)
                def _():
                    load_xyz(it + 1, 1 - par)
                    stage1(1 - par)
                    for d in gather_descs(1 - par):
                        d.start()

                for d in gather_descs(par):
                    d.wait()

                @pl.when(it >= 2)
                def _():
                    for d in out_descs(it - 2, par):
                        d.wait()

                stage3(par)
                for d in out_descs(it, par):
                    d.start()

        for d in out_descs(iters - 2, 0):
            d.wait()
        for d in out_descs(iters - 1, 1):
            d.wait()

    return body


def kernel(xyz, grid):
    n_pts = xyz.shape[0]
    c = grid.shape[1]
    sizes = grid.shape[2:]
    nvox = sizes[0] * sizes[1] * sizes[2]
    mesh = plsc.VectorSubcoreMesh(core_axis_name="c", subcore_axis_name="s")

    gflat = _detile_kernel(grid)

    pack = pl.kernel(
        _make_pack_body(nvox, c),
        out_type=jax.ShapeDtypeStruct((nvox // 4, 16), jnp.float32),
        mesh=mesh,
        compiler_params=_SC_PARAMS,
        scratch_types=[
            pltpu.VMEM((2, c, PF // 128, 128), jnp.float32),
            pltpu.VMEM((2, PF * c // 16, 16), jnp.float32),
            pltpu.SemaphoreType.DMA,
            pltpu.SemaphoreType.DMA,
            pltpu.SemaphoreType.DMA,
            pltpu.SemaphoreType.DMA,
        ],
    )
    table = pack(gflat)

    xyz128 = xyz.T.reshape(n_pts * 3 // 128, 128)
    main = pl.kernel(
        _make_main_body(n_pts, c, sizes),
        out_type=jax.ShapeDtypeStruct((n_pts * c // 128, 128), jnp.float32),
        mesh=mesh,
        compiler_params=_SC_PARAMS,
        scratch_types=[
            pltpu.VMEM((2, B // 128, 128), jnp.float32),
            pltpu.VMEM((2, B // 128, 128), jnp.float32),
            pltpu.VMEM((2, B // 128, 128), jnp.float32),
            pltpu.VMEM((2, B), jnp.float32),
            pltpu.VMEM((2, B), jnp.float32),
            pltpu.VMEM((2, B), jnp.float32),
            pltpu.VMEM((2, B), jnp.int32),
            pltpu.VMEM((2, B), jnp.int32),
            pltpu.VMEM((2, 8, B), jnp.int32),
            pltpu.VMEM((2, 8, B, 16), jnp.float32),
            pltpu.VMEM((2, B * 4 // 128, 128), jnp.float32),
            pltpu.SemaphoreType.DMA,
            pltpu.SemaphoreType.DMA,
            pltpu.SemaphoreType.DMA,
            pltpu.SemaphoreType.DMA,
        ],
    )
    out128 = main(xyz128, table)
    return out128.reshape(c, n_pts).T


# speedup vs baseline: 31.2074x; 1.0017x over previous
import jax
import jax.numpy as jnp
from jax import lax
from jax.experimental import pallas as pl
from jax.experimental.pallas import tpu as pltpu
from jax.experimental.pallas import tpu_sc as plsc

L = 16
NC = 2
NS = 16
NW = NC * NS
B = 256
PF = 4096

XY_CORNERS = [(0, 0), (0, 1), (1, 0), (1, 1)]

_SC_PARAMS = pltpu.CompilerParams(
    needs_layout_passes=False, use_tc_tiling_on_sc=False)


def _detile_kernel(grid):
    _, c, sx, sy, sz = grid.shape
    rows = c * sx * sy * sz // 128
    rows_per_blk = sy * sz // 128

    xb = 4
    def body(g_ref, o_ref):
        o_ref[...] = g_ref[...].reshape(o_ref.shape)

    return pl.pallas_call(
        body,
        grid=(c * sx // xb,),
        in_specs=[pl.BlockSpec((1, 1, xb, sy, sz),
                               lambda i: (0, i // (256 // xb),
                                          i % (256 // xb), 0, 0))],
        out_specs=pl.BlockSpec((rows_per_blk * xb, 128), lambda i: (i, 0)),
        out_shape=jax.ShapeDtypeStruct((rows, 128), jnp.float32),
    )(grid)


def _make_pack_body(nvox, c):
    npw = nvox // NW
    n_chunks = npw // PF
    rows_out = PF * c // 16
    cin_rows = PF // 128
    ch_rows = nvox // 128

    def body(gflat_hbm, table_hbm, cin, cout,
             lsem0, lsem1, ssem0, ssem1):
        wid = lax.axis_index("c") * NS + lax.axis_index("s")
        lanes = lax.iota(jnp.int32, L)
        rep4 = lax.shift_right_logical(lanes, 2)
        col4 = jnp.bitwise_and(lanes, 3)
        lsems = (lsem0, lsem1)
        ssems = (ssem0, ssem1)

        def load_descs(j, slot):
            f0 = wid * npw + j * PF
            return [
                pltpu.make_async_copy(
                    gflat_hbm.at[pl.ds(ch * ch_rows + f0 // 128, cin_rows)],
                    cin.at[slot, ch], lsems[slot])
                for ch in range(c)
            ]

        def store_desc(j, slot):
            f0 = wid * npw + j * PF
            return pltpu.make_async_copy(
                cout.at[slot], table_hbm.at[pl.ds(f0 // 4, rows_out)],
                ssems[slot])

        for d in load_descs(0, 0):
            d.start()

        @pl.loop(0, n_chunks // 2)
        def _pair(p):
            for par in (0, 1):
                j = p * 2 + par
                for d in load_descs(j, par):
                    d.wait()

                @pl.when(j + 1 < n_chunks)
                def _():
                    for d in load_descs(j + 1, 1 - par):
                        d.start()

                @pl.when(j >= 2)
                def _():
                    store_desc(j - 2, par).wait()

                @pl.loop(0, rows_out, unroll=16)
                def _row(r):
                    vi = r * 4 + rep4
                    v = plsc.load_gather(
                        cin.at[par], [col4, lax.shift_right_logical(vi, 7),
                                      jnp.bitwise_and(vi, 127)])
                    cout[par, r, :] = v

                store_desc(j, par).start()

        store_desc(n_chunks - 2, 0).wait()
        store_desc(n_chunks - 1, 1).wait()

    return body


def _make_main_body(n_pts, c, sizes):
    sx, sy, sz = sizes
    qz = sz // 4
    row_y = qz
    row_x = sy * qz
    npw = n_pts // NW
    iters = npw // B
    crd_rows = B // 128
    ch_crd_rows = n_pts // 128
    out_rows = B // 128
    ch_out_rows = n_pts // 128
    scales = [(s - 1) * 0.5 for s in sizes]

    def body(xyz_hbm, table_hbm, out_hbm,
             xv, yv, zv, fx, fy, fz, lb0, lb1, idxb, dest, acc,
             gsem0, gsem1, osem0, osem1):
        wid = lax.axis_index("c") * NS + lax.axis_index("s")
        lanes = lax.iota(jnp.int32, L)
        rep4 = lax.shift_right_logical(lanes, 2)
        col4 = jnp.bitwise_and(lanes, 3)
        gsems = (gsem0, gsem1)
        osems = (osem0, osem1)

        def load_xyz(it, sl):
            tile = wid * iters + it
            for d, buf in ((0, xv), (1, yv), (2, zv)):
                pltpu.sync_copy(
                    xyz_hbm.at[pl.ds(d * ch_crd_rows + tile * crd_rows,
                                     crd_rows)], buf.at[sl])

        def stage1(sl):
            @pl.loop(0, B // L, unroll=4)
            def _cg(g):
                s = pl.ds(g * L, L)
                r = lax.shift_right_logical(g, 3)
                o = jnp.bitwise_and(g, 7) * L

                def prep(v, scale, size):
                    f = v * scale + scale
                    f = jnp.maximum(f, 0.0)
                    i = jnp.minimum(f.astype(jnp.int32), size - 2)
                    return i, f - i.astype(jnp.float32)

                ix, fxv = prep(xv[sl, r, pl.ds(o, L)], scales[0], sx)
                iy, fyv = prep(yv[sl, r, pl.ds(o, L)], scales[1], sy)
                iz, fzv = prep(zv[sl, r, pl.ds(o, L)], scales[2], sz)
                fx[sl, s] = fxv
                fy[sl, s] = fyv
                fz[sl, s] = fzv
                iz1 = iz + 1
                lb0[sl, s] = jnp.bitwise_and(iz, 3) * 4
                lb1[sl, s] = jnp.bitwise_and(iz1, 3) * 4
                xyb = ix * row_x + iy * row_y
                h0 = xyb + lax.shift_right_logical(iz, 2)
                h1 = xyb + lax.shift_right_logical(iz1, 2)
                for k, (cx, cy) in enumerate(XY_CORNERS):
                    off = cx * row_x + cy * row_y
                    idxb[sl, k, s] = h0 + off
                    idxb[sl, 4 + k, s] = h1 + off

        def gather_descs(sl):
            return [
                pltpu.make_async_copy(table_hbm.at[idxb.at[sl, k]],
                                      dest.at[sl, k], gsems[sl])
                for k in range(8)
            ]

        def out_descs(it, sl):
            tile = wid * iters + it
            return [
                pltpu.make_async_copy(
                    acc.at[sl, pl.ds(cc * out_rows, out_rows)],
                    out_hbm.at[pl.ds(cc * ch_out_rows + tile * out_rows,
                                     out_rows)],
                    osems[sl])
                for cc in range(c)
            ]

        def stage3(sl):
            @pl.loop(0, B // 4, unroll=8)
            def _acc(g2):
                ridx = rep4 + g2 * 4
                fxr = plsc.load_gather(fx.at[sl], [ridx])
                fyr = plsc.load_gather(fy.at[sl], [ridx])
                fzr = plsc.load_gather(fz.at[sl], [ridx])
                l0r = plsc.load_gather(lb0.at[sl], [ridx]) + col4
                l1r = plsc.load_gather(lb1.at[sl], [ridx]) + col4
                gz0 = 1.0 - fzr
                wxy = {
                    (0, 0): (1.0 - fxr) * (1.0 - fyr),
                    (0, 1): (1.0 - fxr) * fyr,
                    (1, 0): fxr * (1.0 - fyr),
                    (1, 1): fxr * fyr,
                }
                acc_v = None
                for k, (cx, cy) in enumerate(XY_CORNERS):
                    d0 = plsc.load_gather(dest.at[sl, k], [ridx, l0r])
                    d1 = plsc.load_gather(dest.at[sl, 4 + k], [ridx, l1r])
                    contrib = wxy[(cx, cy)] * (d0 * gz0 + d1 * fzr)
                    acc_v = contrib if acc_v is None else acc_v + contrib
                eo = col4 * B + ridx
                plsc.store_scatter(
                    acc.at[sl], [lax.shift_right_logical(eo, 7),
                                 jnp.bitwise_and(eo, 127)], acc_v)

        load_xyz(0, 0)
        stage1(0)
        for d in gather_descs(0):
            d.start()

        @pl.loop(0, iters // 2)
        def _pair(p):
            for par in (0, 1):
                it = p * 2 + par

                @pl.when(it + 1 < iters)
                def _():
                    load_xyz(it + 1, 1 - par)
                    stage1(1 - par)
                    for d in gather_descs(1 - par):
                        d.start()

                for d in gather_descs(par):
                    d.wait()

                @pl.when(it >= 2)
                def _():
                    for d in out_descs(it - 2, par):
                        d.wait()

                stage3(par)
                for d in out_descs(it, par):
                    d.start()

        for d in out_descs(iters - 2, 0):
            d.wait()
        for d in out_descs(iters - 1, 1):
            d.wait()

    return body


def kernel(xyz, grid):
    n_pts = xyz.shape[0]
    c = grid.shape[1]
    sizes = grid.shape[2:]
    nvox = sizes[0] * sizes[1] * sizes[2]
    mesh = plsc.VectorSubcoreMesh(core_axis_name="c", subcore_axis_name="s")

    gflat = _detile_kernel(grid)

    pack = pl.kernel(
        _make_pack_body(nvox, c),
        out_type=jax.ShapeDtypeStruct((nvox // 4, 16), jnp.float32),
        mesh=mesh,
        compiler_params=_SC_PARAMS,
        scratch_types=[
            pltpu.VMEM((2, c, PF // 128, 128), jnp.float32),
            pltpu.VMEM((2, PF * c // 16, 16), jnp.float32),
            pltpu.SemaphoreType.DMA,
            pltpu.SemaphoreType.DMA,
            pltpu.SemaphoreType.DMA,
            pltpu.SemaphoreType.DMA,
        ],
    )
    table = pack(gflat)

    xyz128 = xyz.T.reshape(n_pts * 3 // 128, 128)
    main = pl.kernel(
        _make_main_body(n_pts, c, sizes),
        out_type=jax.ShapeDtypeStruct((n_pts * c // 128, 128), jnp.float32),
        mesh=mesh,
        compiler_params=_SC_PARAMS,
        scratch_types=[
            pltpu.VMEM((2, B // 128, 128), jnp.float32),
            pltpu.VMEM((2, B // 128, 128), jnp.float32),
            pltpu.VMEM((2, B // 128, 128), jnp.float32),
            pltpu.VMEM((2, B), jnp.float32),
            pltpu.VMEM((2, B), jnp.float32),
            pltpu.VMEM((2, B), jnp.float32),
            pltpu.VMEM((2, B), jnp.int32),
            pltpu.VMEM((2, B), jnp.int32),
            pltpu.VMEM((2, 8, B), jnp.int32),
            pltpu.VMEM((2, 8, B, 16), jnp.float32),
            pltpu.VMEM((2, B * 4 // 128, 128), jnp.float32),
            pltpu.SemaphoreType.DMA,
            pltpu.SemaphoreType.DMA,
            pltpu.SemaphoreType.DMA,
            pltpu.SemaphoreType.DMA,
        ],
    )
    out128 = main(xyz128, table)
    return out128.reshape(c, n_pts).T
